# baseline (speedup 1.0000x reference)
import functools

import jax
import jax.numpy as jnp
from jax import lax
from jax.experimental import pallas as pl
from jax.experimental.pallas import tpu as pltpu
from jax.experimental.pallas import tpu_sc as plsc

_B = 4096
_BOX_DIM = 256
_BOX_OUT = 64
_V_DISC = 100000
_D_DISC = 32
_NF = 26
_V_MD = 1000
_D_MD = 16

_NC = 2
_NS = 16
_NW = _NC * _NS
_BPW = _B // _NW
_MD_CHUNK = 128
_MD_NCHUNK = _NF * _BPW // _MD_CHUNK
_MD_PW = _NF * _BPW


def _sc_disc_body(idx_hbm, tab_hbm, out_hbm, idx_v, rows_v, sem, osem):
    wid = lax.axis_index("s") * _NC + lax.axis_index("c")
    pltpu.sync_copy(idx_hbm.at[pl.ds(wid * _BPW, _BPW)], idx_v)
    for g in range(_BPW // 16):
        v = idx_v[pl.ds(g * 16, 16)]
        for l in range(16):
            j = g * 16 + l
            pltpu.make_async_copy(
                tab_hbm.at[pl.ds(v[l], 1)], rows_v.at[pl.ds(j, 1)],
                sem).start()

    def drain(j, carry):
        pltpu.make_async_copy(
            tab_hbm.at[pl.ds(0, 1)], rows_v.at[pl.ds(j, 1)], sem).wait()
        return carry

    lax.fori_loop(0, _BPW, drain, 0, unroll=8)
    pltpu.async_copy(rows_v, out_hbm.at[wid], osem).wait()


@functools.cache
def _sc_disc():
    return pl.kernel(
        _sc_disc_body,
        mesh=plsc.VectorSubcoreMesh(core_axis_name="c", subcore_axis_name="s"),
        out_type=jax.ShapeDtypeStruct((_NW, _BPW, _D_DISC), jnp.float32),
        scratch_types=[
            pltpu.VMEM((_BPW,), jnp.int32),
            pltpu.VMEM((_BPW, _D_DISC), jnp.float32),
            pltpu.SemaphoreType.DMA,
            pltpu.SemaphoreType.DMA,
        ],
    )


def _sc_md_body(idx_hbm, tab_hbm, out_hbm, idx_v, rows_v, sem, osem):
    wid = lax.axis_index("s") * _NC + lax.axis_index("c")
    tab = tab_hbm
    pltpu.sync_copy(idx_hbm.at[pl.ds(wid * _MD_PW, _MD_PW)], idx_v)
    cps = []
    for c in range(_MD_NCHUNK):
        cps.append(pltpu.async_copy(
            tab.at[idx_v.at[pl.ds(c * _MD_CHUNK, _MD_CHUNK)]],
            rows_v.at[c], sem))
    for cp in cps:
        cp.wait()
    pltpu.async_copy(rows_v, out_hbm.at[wid], osem).wait()


@functools.cache
def _sc_md():
    return pl.kernel(
        _sc_md_body,
        mesh=plsc.VectorSubcoreMesh(core_axis_name="c", subcore_axis_name="s"),
        out_type=jax.ShapeDtypeStruct((_NW, _MD_NCHUNK, _MD_CHUNK, _D_MD),
                                      jnp.float32),
        scratch_types=[
            pltpu.VMEM((_MD_PW,), jnp.int32),
            pltpu.VMEM((_MD_NCHUNK, _MD_CHUNK, _D_MD), jnp.float32),
            pltpu.SemaphoreType.DMA,
            pltpu.SemaphoreType.DMA,
        ],
        compiler_params=pltpu.CompilerParams(use_tc_tiling_on_sc=False),
    )


def _tc_assemble_body(obs_ref, w_ref, b_ref, disc_ref, md_ref, out_ref):
    acc = jnp.dot(obs_ref[...], w_ref[...], preferred_element_type=jnp.float32)
    acc = acc + b_ref[...]
    out_ref[...] = jnp.concatenate([acc, disc_ref[...], md_ref[...]], axis=-1)


_BM = 512

_tc_assemble = pl.pallas_call(
    _tc_assemble_body,
    grid=(_B // _BM,),
    in_specs=[
        pl.BlockSpec((_BM, _BOX_DIM), lambda i: (i, 0)),
        pl.BlockSpec((_BOX_DIM, _BOX_OUT), lambda i: (0, 0)),
        pl.BlockSpec((1, _BOX_OUT), lambda i: (0, 0)),
        pl.BlockSpec((_BM, _D_DISC), lambda i: (i, 0)),
        pl.BlockSpec((_BM, _NF * _D_MD), lambda i: (i, 0)),
    ],
    out_specs=pl.BlockSpec((_BM, _BOX_OUT + _D_DISC + _NF * _D_MD),
                           lambda i: (i, 0)),
    out_shape=jax.ShapeDtypeStruct(
        (_B, _BOX_OUT + _D_DISC + _NF * _D_MD), jnp.float32),
)


def kernel(obs_box, obs_discrete, obs_multidiscrete, W_box, b_box,
           emb_discrete, emb_multi):
    offs = (jnp.arange(_NF, dtype=jnp.int32) * _V_MD)[None, :]
    idx_md = (obs_multidiscrete + offs).reshape(-1)
    tab_md_wide = jax.lax.optimization_barrier(
        emb_multi.reshape(_NF * _V_MD * _D_MD // 128, 128))
    tab_md = tab_md_wide.reshape(_NF * _V_MD, _D_MD)

    f_disc = _sc_disc()(obs_discrete, emb_discrete)
    f_md = _sc_md()(idx_md, tab_md)
    f_disc = f_disc.reshape(_B, _D_DISC)
    f_md = f_md.reshape(_B, _NF * _D_MD)

    return _tc_assemble(obs_box, W_box, b_box.reshape(1, _BOX_OUT),
                        f_disc, f_md)


# speedup vs baseline: 1.0703x; 1.0703x over previous
import functools

import jax
import jax.numpy as jnp
from jax import lax
from jax.experimental import pallas as pl
from jax.experimental.pallas import tpu as pltpu
from jax.experimental.pallas import tpu_sc as plsc

_B = 4096
_BOX_DIM = 256
_BOX_OUT = 64
_V_DISC = 100000
_D_DISC = 32
_NF = 26
_V_MD = 1000
_D_MD = 16
_NFP = 32

_NC = 2
_NS = 16
_NW = _NC * _NS
_BPW = _B // _NW
_MD_CHUNK = 128
_MD_PW = _NFP * _BPW
_MD_NCHUNK = _MD_PW // _MD_CHUNK


def _sc_disc_body(idx_hbm, tab_hbm, out_hbm, idx_v, rows_v, sem, osem):
    wid = lax.axis_index("s") * _NC + lax.axis_index("c")
    pltpu.sync_copy(idx_hbm.at[pl.ds(wid * _BPW, _BPW)], idx_v)
    for g in range(_BPW // 16):
        v = idx_v[pl.ds(g * 16, 16)]
        for l in range(16):
            j = g * 16 + l
            pltpu.make_async_copy(
                tab_hbm.at[pl.ds(v[l], 1)], rows_v.at[pl.ds(j, 1)],
                sem).start()

    def drain(j, carry):
        pltpu.make_async_copy(
            tab_hbm.at[pl.ds(0, 1)], rows_v.at[pl.ds(j, 1)], sem).wait()
        return carry

    lax.fori_loop(0, _BPW, drain, 0, unroll=8)
    pltpu.async_copy(rows_v, out_hbm.at[wid], osem).wait()


@functools.cache
def _sc_disc():
    return pl.kernel(
        _sc_disc_body,
        mesh=plsc.VectorSubcoreMesh(core_axis_name="c", subcore_axis_name="s"),
        out_type=jax.ShapeDtypeStruct((_NW, _BPW, _D_DISC), jnp.float32),
        scratch_types=[
            pltpu.VMEM((_BPW,), jnp.int32),
            pltpu.VMEM((_BPW, _D_DISC), jnp.float32),
            pltpu.SemaphoreType.DMA,
            pltpu.SemaphoreType.DMA,
        ],
    )


def _sc_md_body(idx_hbm, tab_hbm, out_hbm, idx_v, rows_v, sem, osem):
    wid = lax.axis_index("s") * _NC + lax.axis_index("c")
    pltpu.sync_copy(idx_hbm.at[pl.ds(wid * _MD_PW, _MD_PW)], idx_v)
    cps = []
    for c in range(_MD_NCHUNK):
        cps.append(pltpu.async_copy(
            tab_hbm.at[idx_v.at[pl.ds(c * _MD_CHUNK, _MD_CHUNK)]],
            rows_v.at[c], sem))
    for cp in cps:
        cp.wait()
    pltpu.async_copy(rows_v, out_hbm.at[wid], osem).wait()


@functools.cache
def _sc_md():
    return pl.kernel(
        _sc_md_body,
        mesh=plsc.VectorSubcoreMesh(core_axis_name="c", subcore_axis_name="s"),
        out_type=jax.ShapeDtypeStruct((_NW, _MD_NCHUNK, _MD_CHUNK, _D_MD),
                                      jnp.float32),
        scratch_types=[
            pltpu.VMEM((_MD_PW,), jnp.int32),
            pltpu.VMEM((_MD_NCHUNK, _MD_CHUNK, _D_MD), jnp.float32),
            pltpu.SemaphoreType.DMA,
            pltpu.SemaphoreType.DMA,
        ],
        compiler_params=pltpu.CompilerParams(use_tc_tiling_on_sc=False),
    )


def _tc_assemble_body(obs_ref, w_ref, b_ref, disc_ref, md_ref, out_ref):
    acc = jnp.dot(obs_ref[...], w_ref[...], preferred_element_type=jnp.float32)
    acc = acc + b_ref[...]
    md = md_ref[...].reshape(_BM, _NFP * _D_MD)[:, : _NF * _D_MD]
    out_ref[...] = jnp.concatenate([acc, disc_ref[...], md], axis=-1)


_BM = 512
_MDR = _NFP * _D_MD // 128

_tc_assemble = pl.pallas_call(
    _tc_assemble_body,
    grid=(_B // _BM,),
    in_specs=[
        pl.BlockSpec((_BM, _BOX_DIM), lambda i: (i, 0)),
        pl.BlockSpec((_BOX_DIM, _BOX_OUT), lambda i: (0, 0)),
        pl.BlockSpec((1, _BOX_OUT), lambda i: (0, 0)),
        pl.BlockSpec((_BM, _D_DISC), lambda i: (i, 0)),
        pl.BlockSpec((_BM * _MDR, 128), lambda i: (i, 0)),
    ],
    out_specs=pl.BlockSpec((_BM, _BOX_OUT + _D_DISC + _NF * _D_MD),
                           lambda i: (i, 0)),
    out_shape=jax.ShapeDtypeStruct(
        (_B, _BOX_OUT + _D_DISC + _NF * _D_MD), jnp.float32),
)


def kernel(obs_box, obs_discrete, obs_multidiscrete, W_box, b_box,
           emb_discrete, emb_multi):
    f = jnp.arange(_NFP, dtype=jnp.int32)[None, :]
    offs = jnp.where(f < _NF, f * _V_MD, 0)
    b = jnp.arange(_B, dtype=jnp.int32)[:, None]
    pad_idx = (b * (_NFP - _NF) + (f - _NF)) % (_NF * _V_MD)
    idx32 = jnp.where(
        f < _NF,
        jnp.pad(obs_multidiscrete, ((0, 0), (0, _NFP - _NF))) + offs,
        pad_idx,
    ).reshape(-1)
    tab_md = emb_multi.reshape(_NF * _V_MD, _D_MD)

    f_disc = _sc_disc()(obs_discrete, emb_discrete)
    f_md = _sc_md()(idx32, tab_md)
    f_disc = f_disc.reshape(_B, _D_DISC)
    f_md = f_md.reshape(_B * _MDR, 128)

    return _tc_assemble(obs_box, W_box, b_box.reshape(1, _BOX_OUT),
                        f_disc, f_md)
